# baseline (speedup 1.0000x reference)
import functools

import jax
import jax.numpy as jnp
from jax import lax
from jax.experimental import pallas as pl
from jax.experimental.pallas import tpu as pltpu
from jax.experimental.pallas import tpu_sc as plsc

B, M, D = 1024, 2048, 16
BR = 128
NT = 32
RPT = B // NT
NR = B * D * M // 128


def _tc_body(times_ref, mask_ref, aux_ref, tout_ref, mout_ref, pos_ref):
    m = mask_ref[...]
    cnt = jnp.sum(m.astype(jnp.int32), axis=1, keepdims=True)
    acc = aux_ref[:, 1:2] > 0.5
    canw = acc & (cnt < M)
    safe = jnp.minimum(cnt, M - 1)
    lanes = jax.lax.broadcasted_iota(jnp.int32, (BR, M), 1)
    sel = (lanes == safe) & canw
    tout_ref[...] = jnp.where(sel, aux_ref[:, 0:1], times_ref[...])
    mout_ref[...] = m | sel.astype(jnp.int8)
    pos_ref[...] = jnp.broadcast_to(jnp.where(canw, safe, -1), (BR, 128))


def _sc_patch(pos_hbm, markf_hbm, mref, posv, markv, curv, valv, sem):
    c = lax.axis_index("c")
    s = lax.axis_index("s")
    wid = s * 2 + c
    base = wid * RPT
    pltpu.sync_copy(pos_hbm.at[pl.ds(base * 1, RPT)], posv)
    for d in range(D):
        pltpu.sync_copy(markf_hbm.at[pl.ds(d * B + base, RPT)],
                        markv.at[pl.ds(d * RPT, RPT)])
    iota = lax.iota(jnp.int32, 16)
    idxs = {}
    handles = []
    for h in range(RPT // 16):
        pv = posv[pl.ds(h * 16, 16)]
        sp = jnp.maximum(pv, 0)
        bvec = base + h * 16 + iota
        for d in range(D):
            idx = ((2 * bvec + (d // 8)) * 16384
                   + ((sp >> 7) << 10) + (d % 8) * 128 + (sp & 127))
            idxs[(h, d)] = idx
            handles.append(pltpu.async_copy(
                mref.at[idx], curv.at[pl.ds((h * D + d) * 16, 16)], sem))
    for hd in handles:
        hd.wait()
    for h in range(RPT // 16):
        keep = posv[pl.ds(h * 16, 16)] < 0
        for d in range(D):
            o = (h * D + d) * 16
            mvals = markv[pl.ds(d * RPT + h * 16, 16)]
            cvals = curv[pl.ds(o, 16)]
            valv[pl.ds(o, 16)] = jnp.where(keep, cvals, mvals)
    handles = []
    for h in range(RPT // 16):
        for d in range(D):
            handles.append(pltpu.async_copy(
                valv.at[pl.ds((h * D + d) * 16, 16)], mref.at[idxs[(h, d)]],
                sem))
    for hd in handles:
        hd.wait()


def kernel(times, mask, marks, t, mark, accepted):
    mask8 = mask.astype(jnp.int8)
    mark_tflat = jnp.reshape(jnp.transpose(mark, (1, 0)), (D * B,))
    aux = jnp.concatenate(
        [t[:, None], accepted.astype(jnp.float32)[:, None],
         jnp.zeros((B, 126), jnp.float32)], axis=1)

    new_times, new_mask8, pos2d = pl.pallas_call(
        _tc_body,
        grid=(B // BR,),
        in_specs=[
            pl.BlockSpec((BR, M), lambda i: (i, 0)),
            pl.BlockSpec((BR, M), lambda i: (i, 0)),
            pl.BlockSpec((BR, 128), lambda i: (i, 0)),
        ],
        out_specs=[
            pl.BlockSpec((BR, M), lambda i: (i, 0)),
            pl.BlockSpec((BR, M), lambda i: (i, 0)),
            pl.BlockSpec((BR, 128), lambda i: (i, 0)),
        ],
        out_shape=[
            jax.ShapeDtypeStruct((B, M), jnp.float32),
            jax.ShapeDtypeStruct((B, M), jnp.int8),
            jax.ShapeDtypeStruct((B, 128), jnp.int32),
        ],
    )(times, mask8, aux)
    pos1d = pos2d[:, 0]

    x = jnp.transpose(marks, (0, 2, 1)).reshape(B * D, M)
    x = x.reshape(B * D // 8, 8, M // 128, 128)
    x = jnp.transpose(x, (0, 2, 1, 3))
    marks_lin = x.reshape(NR, 128).reshape(NR * 128)
    mref = jax.new_ref(marks_lin)

    mesh = plsc.VectorSubcoreMesh(core_axis_name="c", subcore_axis_name="s")
    sc_patch = functools.partial(
        pl.kernel,
        out_type=(),
        mesh=mesh,
        scratch_types=[
            pltpu.VMEM((RPT,), jnp.int32),
            pltpu.VMEM((RPT * D,), jnp.float32),
            pltpu.VMEM((RPT * D,), jnp.float32),
            pltpu.VMEM((RPT * D,), jnp.float32),
            pltpu.SemaphoreType.DMA,
        ],
    )(_sc_patch)
    sc_patch(pos1d, mark_tflat, mref)

    y = mref[...].reshape(NR, 128).reshape(B * D // 8, M // 128, 8, 128)
    y = jnp.transpose(y, (0, 2, 1, 3))
    new_marks = jnp.transpose(y.reshape(B, D, M), (0, 2, 1))
    return new_times, new_mask8.astype(jnp.bool_), new_marks


# speedup vs baseline: 1.3665x; 1.3665x over previous
import jax
import jax.numpy as jnp
from jax.experimental import pallas as pl

B, M, D = 1024, 2048, 16
BR = 128
MQ = 512
NQ = M // MQ


def _body(times_ref, mask_ref, aux_ref, markt_ref, marks_ref,
          tout_ref, mout_ref, marksout_ref):
    m = mask_ref[...]
    cnt = jnp.sum(m.astype(jnp.int32), axis=1, keepdims=True)
    acc = aux_ref[:, 1:2] > 0.5
    canw = acc & (cnt < M)
    safe = jnp.minimum(cnt, M - 1)
    lanes = jax.lax.broadcasted_iota(jnp.int32, (BR, M), 1)
    sel = (lanes == safe) & canw
    tout_ref[...] = jnp.where(sel, aux_ref[:, 0:1], times_ref[...])
    mout_ref[...] = m | sel.astype(jnp.int8)
    i = pl.program_id(0)
    q = pl.program_id(1)
    mt = markt_ref[:, pl.ds(pl.multiple_of(i * BR, BR), BR)]
    markcol = jnp.transpose(mt, (1, 0))[:, :, None]
    qlanes = (jax.lax.broadcasted_iota(jnp.int32, (BR, MQ), 1)
              + q * MQ)
    qsel = (qlanes == safe) & canw
    marksout_ref[...] = jnp.where(qsel[:, None, :], markcol,
                                  marks_ref[...])


def kernel(times, mask, marks, t, mark, accepted):
    marks_t = jnp.transpose(marks, (0, 2, 1))
    mark_t = jnp.transpose(mark, (1, 0))
    mask8 = mask.astype(jnp.int8)
    aux = jnp.concatenate(
        [t[:, None], accepted.astype(jnp.float32)[:, None],
         jnp.zeros((B, 126), jnp.float32)], axis=1)

    grid = (B // BR, NQ)
    new_times, new_mask8, new_marks_t = pl.pallas_call(
        _body,
        grid=grid,
        in_specs=[
            pl.BlockSpec((BR, M), lambda i, q: (i, 0)),
            pl.BlockSpec((BR, M), lambda i, q: (i, 0)),
            pl.BlockSpec((BR, 128), lambda i, q: (i, 0)),
            pl.BlockSpec((D, B), lambda i, q: (0, 0)),
            pl.BlockSpec((BR, D, MQ), lambda i, q: (i, 0, q)),
        ],
        out_specs=[
            pl.BlockSpec((BR, M), lambda i, q: (i, 0)),
            pl.BlockSpec((BR, M), lambda i, q: (i, 0)),
            pl.BlockSpec((BR, D, MQ), lambda i, q: (i, 0, q)),
        ],
        out_shape=[
            jax.ShapeDtypeStruct((B, M), jnp.float32),
            jax.ShapeDtypeStruct((B, M), jnp.int8),
            jax.ShapeDtypeStruct((B, D, M), jnp.float32),
        ],
    )(times, mask8, aux, mark_t, marks_t)
    return (new_times, new_mask8.astype(jnp.bool_),
            jnp.transpose(new_marks_t, (0, 2, 1)))
